# baseline (speedup 1.0000x reference)
import jax
import jax.numpy as jnp
from jax.experimental import pallas as pl
from jax.experimental.pallas import tpu as pltpu

EMBED_DIM = 1024
NUM_EXPERTS = 64
ACTIVE_EXPERTS = 2
HIDDEN = 4 * EMBED_DIM
NTOK = 8
NPAIR = ACTIVE_EXPERTS * NTOK

BH = 1024
NH = HIDDEN // BH


def _router_body(x_ref, wr_ref, br_ref, eidx_ref, tok_ref, wts_ref):
    scores = jnp.maximum(
        jnp.dot(x_ref[...], wr_ref[...], preferred_element_type=jnp.float32)
        + br_ref[...],
        0.0,
    )
    i0 = jnp.argmax(scores, axis=1)
    v0 = jnp.max(scores, axis=1)
    col = jax.lax.broadcasted_iota(jnp.int32, scores.shape, 1)
    masked = jnp.where(col == i0[:, None], -jnp.inf, scores)
    i1 = jnp.argmax(masked, axis=1)
    v1 = jnp.max(masked, axis=1)
    e1 = jnp.exp(v1 - v0)
    denom = 1.0 + e1
    w0 = 1.0 / denom
    w1 = e1 / denom

    eidx_row = jnp.concatenate([i0[None, :], i1[None, :]], axis=1)
    wts_row = jnp.concatenate([w0[None, :], w1[None, :]], axis=1)
    pid_row = jax.lax.broadcasted_iota(jnp.int32, (1, NPAIR), 1)
    tok_row = pid_row % NTOK

    pid_col = jax.lax.broadcasted_iota(jnp.int32, (NPAIR, 1), 0)
    eidx_col = jnp.concatenate([i0[:, None], i1[:, None]], axis=0)
    key_col = eidx_col * NPAIR + pid_col
    key_row = eidx_row * NPAIR + pid_row
    lt = (key_col < key_row).astype(jnp.int32)
    rank_row = jnp.sum(lt, axis=0, keepdims=True)
    rr = jax.lax.broadcasted_iota(jnp.int32, (NPAIR, NPAIR), 0)
    P = (rr == rank_row).astype(jnp.int32)
    eidx_ref[...] = jnp.sum(P * eidx_row, axis=1, keepdims=True)
    tok_ref[...] = jnp.sum(P * tok_row, axis=1, keepdims=True)
    wts_ref[...] = jnp.sum(P.astype(jnp.float32) * wts_row, axis=1,
                           keepdims=True)


def _expert_body(eidx_ref, tok_ref, wts_ref, x_ref, w1a_ref, w1b_ref, b1_ref,
                 w2a_ref, w2b_ref, b2_ref, out_ref, acc_ref):
    h = pl.program_id(0)
    p = pl.program_id(1)
    w = wts_ref[p]
    t = tok_ref[p]

    @pl.when(jnp.logical_and(h == 0, p == 0))
    def _init():
        acc_ref[...] = jnp.zeros_like(acc_ref)

    xa = x_ref[0, :, : EMBED_DIM // 2]
    xb = x_ref[0, :, EMBED_DIM // 2:]
    hid = jnp.maximum(
        jnp.dot(xa, w1a_ref[0, 0], preferred_element_type=jnp.float32)
        + jnp.dot(xb, w1b_ref[0, 0], preferred_element_type=jnp.float32)
        + b1_ref[0],
        0.0,
    )
    part = (
        jnp.dot(hid[:, : BH // 2], w2a_ref[0, 0],
                preferred_element_type=jnp.float32)
        + jnp.dot(hid[:, BH // 2:], w2b_ref[0, 0],
                  preferred_element_type=jnp.float32)
    )

    contrib = w * part

    @pl.when(h == 0)
    def _bias():
        acc_ref[pl.ds(t, 1), :] += w * b2_ref[0] + contrib

    @pl.when(h != 0)
    def _acc():
        acc_ref[pl.ds(t, 1), :] += contrib

    @pl.when(jnp.logical_and(h == NH - 1, p == NPAIR - 1))
    def _flush():
        out_ref[...] = acc_ref[...].reshape(out_ref.shape)


@jax.jit
def kernel(x, Wr, br, W1, b1, W2, b2):
    x2 = x.reshape(NTOK, EMBED_DIM)

    eidx, tok, wts = pl.pallas_call(
        _router_body,
        out_shape=(
            jax.ShapeDtypeStruct((NPAIR, 1), jnp.int32),
            jax.ShapeDtypeStruct((NPAIR, 1), jnp.int32),
            jax.ShapeDtypeStruct((NPAIR, 1), jnp.float32),
        ),
    )(x2, Wr, br.reshape(1, NUM_EXPERTS))

    eidx = eidx.reshape(NPAIR)
    tok = tok.reshape(NPAIR)
    wts = wts.reshape(NPAIR)

    grid_spec = pltpu.PrefetchScalarGridSpec(
        num_scalar_prefetch=3,
        grid=(NH, NPAIR),
        in_specs=[
            pl.BlockSpec((1, 1, EMBED_DIM),
                         lambda h, p, eidx, tok, wts: (tok[p], 0, 0)),
            pl.BlockSpec((1, 1, EMBED_DIM // 2, BH),
                         lambda h, p, eidx, tok, wts: (eidx[p], 0, 0, h)),
            pl.BlockSpec((1, 1, EMBED_DIM // 2, BH),
                         lambda h, p, eidx, tok, wts: (eidx[p], 1, 0, h)),
            pl.BlockSpec((1, 1, BH),
                         lambda h, p, eidx, tok, wts: (eidx[p], 0, h)),
            pl.BlockSpec((1, 1, BH // 2, EMBED_DIM),
                         lambda h, p, eidx, tok, wts: (eidx[p], 2 * h, 0, 0)),
            pl.BlockSpec((1, 1, BH // 2, EMBED_DIM),
                         lambda h, p, eidx, tok, wts: (eidx[p], 2 * h + 1, 0, 0)),
            pl.BlockSpec((1, 1, EMBED_DIM),
                         lambda h, p, eidx, tok, wts: (eidx[p], 0, 0)),
        ],
        out_specs=pl.BlockSpec((NTOK, 1, EMBED_DIM),
                               lambda h, p, eidx, tok, wts: (0, 0, 0)),
        scratch_shapes=[pltpu.VMEM((NTOK, EMBED_DIM), jnp.float32)],
    )

    out = pl.pallas_call(
        _expert_body,
        grid_spec=grid_spec,
        out_shape=jax.ShapeDtypeStruct((NTOK, 1, EMBED_DIM), jnp.float32),
    )(eidx, tok, wts, x2.reshape(NTOK, 1, EMBED_DIM),
      W1.reshape(NUM_EXPERTS, 2, EMBED_DIM // 2, HIDDEN),
      W1.reshape(NUM_EXPERTS, 2, EMBED_DIM // 2, HIDDEN),
      b1.reshape(NUM_EXPERTS, 1, HIDDEN),
      W2.reshape(NUM_EXPERTS, 2 * NH, BH // 2, EMBED_DIM),
      W2.reshape(NUM_EXPERTS, 2 * NH, BH // 2, EMBED_DIM),
      b2.reshape(NUM_EXPERTS, 1, EMBED_DIM))

    return out.reshape(x.shape)


# speedup vs baseline: 1.0370x; 1.0370x over previous
import jax
import jax.numpy as jnp
from jax.experimental import pallas as pl
from jax.experimental.pallas import tpu as pltpu

EMBED_DIM = 1024
NUM_EXPERTS = 64
ACTIVE_EXPERTS = 2
HIDDEN = 4 * EMBED_DIM
NTOK = 8
NPAIR = ACTIVE_EXPERTS * NTOK

BH = 1024
NH = HIDDEN // BH


def _router_body(x_ref, wr_ref, br_ref, eidx_ref, tok_ref, wts_ref):
    scores = jnp.maximum(
        jnp.dot(x_ref[:, 0, :], wr_ref[...], preferred_element_type=jnp.float32)
        + br_ref[...],
        0.0,
    )
    i0 = jnp.argmax(scores, axis=1)
    v0 = jnp.max(scores, axis=1)
    col = jax.lax.broadcasted_iota(jnp.int32, scores.shape, 1)
    masked = jnp.where(col == i0[:, None], -jnp.inf, scores)
    i1 = jnp.argmax(masked, axis=1)
    v1 = jnp.max(masked, axis=1)
    e1 = jnp.exp(v1 - v0)
    denom = 1.0 + e1
    w0 = 1.0 / denom
    w1 = e1 / denom

    eidx_row = jnp.concatenate([i0[None, :], i1[None, :]], axis=1)
    wts_row = jnp.concatenate([w0[None, :], w1[None, :]], axis=1)
    pid_row = jax.lax.broadcasted_iota(jnp.int32, (1, NPAIR), 1)
    tok_row = pid_row % NTOK

    pid_col = jax.lax.broadcasted_iota(jnp.int32, (NPAIR, 1), 0)
    eidx_col = jnp.concatenate([i0[:, None], i1[:, None]], axis=0)
    key_col = eidx_col * NPAIR + pid_col
    key_row = eidx_row * NPAIR + pid_row
    lt = (key_col < key_row).astype(jnp.int32)
    rank_row = jnp.sum(lt, axis=0, keepdims=True)
    rr = jax.lax.broadcasted_iota(jnp.int32, (NPAIR, NPAIR), 0)
    P = (rr == rank_row).astype(jnp.int32)
    eidx_ref[...] = jnp.sum(P * eidx_row, axis=1, keepdims=True)
    tok_ref[...] = jnp.sum(P * tok_row, axis=1, keepdims=True)
    wts_ref[...] = jnp.sum(P.astype(jnp.float32) * wts_row, axis=1,
                           keepdims=True)


def _expert_body(eidx_ref, tok_ref, wts_ref, x_ref, w1_ref, b1_ref, w2_ref,
                 b2_ref, out_ref, acc_ref):
    h = pl.program_id(0)
    p = pl.program_id(1)
    w = wts_ref[p, 0]
    t = tok_ref[p, 0]

    @pl.when(jnp.logical_and(h == 0, p == 0))
    def _init():
        acc_ref[...] = jnp.zeros_like(acc_ref)

    hid = jnp.maximum(
        jnp.dot(x_ref[0], w1_ref[0], preferred_element_type=jnp.float32)
        + b1_ref[0],
        0.0,
    )
    part = jnp.dot(hid, w2_ref[0], preferred_element_type=jnp.float32)

    contrib = w * part

    @pl.when(h == 0)
    def _bias():
        acc_ref[pl.ds(t, 1), :] += w * b2_ref[0] + contrib

    @pl.when(h != 0)
    def _acc():
        acc_ref[pl.ds(t, 1), :] += contrib

    @pl.when(jnp.logical_and(h == NH - 1, p == NPAIR - 1))
    def _flush():
        out_ref[...] = acc_ref[...].reshape(out_ref.shape)


@jax.jit
def kernel(x, Wr, br, W1, b1, W2, b2):
    eidx, tok, wts = pl.pallas_call(
        _router_body,
        out_shape=(
            jax.ShapeDtypeStruct((NPAIR, 1), jnp.int32),
            jax.ShapeDtypeStruct((NPAIR, 1), jnp.int32),
            jax.ShapeDtypeStruct((NPAIR, 1), jnp.float32),
        ),
    )(x, Wr, br.reshape(1, NUM_EXPERTS))

    grid_spec = pltpu.PrefetchScalarGridSpec(
        num_scalar_prefetch=3,
        grid=(NH, NPAIR),
        in_specs=[
            pl.BlockSpec((1, 1, EMBED_DIM),
                         lambda h, p, eidx, tok, wts: (tok[p, 0], 0, 0)),
            pl.BlockSpec((1, EMBED_DIM, BH),
                         lambda h, p, eidx, tok, wts: (eidx[p, 0], 0, h)),
            pl.BlockSpec((1, 1, BH),
                         lambda h, p, eidx, tok, wts: (eidx[p, 0], 0, h)),
            pl.BlockSpec((1, BH, EMBED_DIM),
                         lambda h, p, eidx, tok, wts: (eidx[p, 0], h, 0)),
            pl.BlockSpec((1, 1, EMBED_DIM),
                         lambda h, p, eidx, tok, wts: (eidx[p, 0], 0, 0)),
        ],
        out_specs=pl.BlockSpec((NTOK, 1, EMBED_DIM),
                               lambda h, p, eidx, tok, wts: (0, 0, 0)),
        scratch_shapes=[pltpu.VMEM((NTOK, EMBED_DIM), jnp.float32)],
    )

    out = pl.pallas_call(
        _expert_body,
        grid_spec=grid_spec,
        out_shape=jax.ShapeDtypeStruct((NTOK, 1, EMBED_DIM), jnp.float32),
    )(eidx, tok, wts, x, W1,
      b1.reshape(NUM_EXPERTS, 1, HIDDEN), W2,
      b2.reshape(NUM_EXPERTS, 1, EMBED_DIM))

    return out
